# baseline (speedup 1.0000x reference)
import functools
import jax
import jax.numpy as jnp
from jax import lax
from jax.experimental import pallas as pl
from jax.experimental.pallas import tpu as pltpu
from jax.experimental.pallas import tpu_sc as plsc

MAX_SIZE = 1000
ROWS_PER_CHUNK = 32
NBUF = 2


def kernel(x):
    B, F = x.shape
    n = B * F
    nc, ns = 2, 16
    nw = nc * ns
    rows_pw = n // nw
    nchunks = rows_pw // ROWS_PER_CHUNK
    chunk_words = ROWS_PER_CHUNK * MAX_SIZE
    mesh = plsc.VectorSubcoreMesh(
        core_axis_name="c", subcore_axis_name="s", num_cores=nc, num_subcores=ns
    )

    @functools.partial(
        pl.kernel,
        mesh=mesh,
        compiler_params=pltpu.CompilerParams(needs_layout_passes=False),
        out_type=jax.ShapeDtypeStruct((n * MAX_SIZE,), jnp.float32),
        scratch_types=[
            pltpu.VMEM((rows_pw,), jnp.int32),
            pltpu.VMEM((chunk_words,), jnp.float32),
            pltpu.VMEM((chunk_words,), jnp.float32),
            pltpu.SemaphoreType.DMA,
            pltpu.SemaphoreType.DMA,
        ],
    )
    def onehot(x_hbm, out_hbm, idx_v, buf0, buf1, sem0, sem1):
        wid = lax.axis_index("s") * nc + lax.axis_index("c")
        row0 = wid * rows_pw
        pltpu.sync_copy(x_hbm.at[pl.ds(row0, rows_pw)], idx_v)

        bufs = (buf0, buf1)
        sems = (sem0, sem1)
        zeros16 = jnp.zeros((16,), jnp.float32)
        ones16 = jnp.ones((16,), jnp.float32)
        iota16 = lax.iota(jnp.int32, 16)

        def zero_body(j, _):
            buf0[pl.ds(j * 16, 16)] = zeros16
            buf1[pl.ds(j * 16, 16)] = zeros16
            return 0

        lax.fori_loop(0, chunk_words // 16, zero_body, 0)

        def offsets(cc, s):
            xv = idx_v[pl.ds(cc * ROWS_PER_CHUNK + s * 16, 16)]
            return (iota16 + s * 16) * MAX_SIZE + xv

        def scatter(cc, buf, vals):
            for s in range(ROWS_PER_CHUNK // 16):
                plsc.store_scatter(buf, [offsets(cc, s)], vals)

        def start_dma(cc, b):
            pltpu.async_copy(
                bufs[b],
                out_hbm.at[pl.ds((row0 + cc * ROWS_PER_CHUNK) * MAX_SIZE, chunk_words)],
                sems[b],
            )

        def wait_dma(b):
            pltpu.make_async_copy(
                bufs[b], out_hbm.at[pl.ds(0, chunk_words)], sems[b]
            ).wait()

        for b in range(NBUF):
            scatter(b, bufs[b], ones16)
            start_dma(b, b)

        def body(i, _):
            for b in range(NBUF):
                cc = NBUF + i * NBUF + b
                wait_dma(b)
                scatter(cc - NBUF, bufs[b], zeros16)
                scatter(cc, bufs[b], ones16)
                start_dma(cc, b)
            return 0

        lax.fori_loop(0, (nchunks - NBUF) // NBUF, body, 0)

        for b in range(NBUF):
            wait_dma(b)

    out = onehot(x.reshape(n))
    return out.reshape(B, F, MAX_SIZE)
